# baseline (speedup 1.0000x reference)
import functools

import jax
import jax.numpy as jnp
from jax import lax
from jax.experimental import pallas as pl
from jax.experimental.pallas import tpu as pltpu
from jax.experimental.pallas import tpu_sc as plsc

_B = 16384
_NF = 128
_HID = 1024
_DT, _DW, _DD = 64, 16, 128
_NCONT = _NF - 3
_DE = _DD + _DT + _DW

_NC, _NS = 2, 16
_NW = _NC * _NS
_BPW = _B // _NW
_CHUNK = 128
_NCH = _BPW // _CHUNK

_BB = 2048
_GRID = _B // _BB


def _sc_gather(t_tab, w_tab, d_tab, idx3):
    mesh = plsc.VectorSubcoreMesh(core_axis_name="c", subcore_axis_name="s")

    @functools.partial(
        pl.kernel,
        mesh=mesh,
        out_type=jax.ShapeDtypeStruct((_B, _DE), jnp.float32),
        scratch_types=[
            pltpu.VMEM((_BPW,), jnp.int32),
            pltpu.VMEM((_BPW,), jnp.int32),
            pltpu.VMEM((_BPW,), jnp.int32),
            pltpu.VMEM((_CHUNK, 128), jnp.float32),
            pltpu.VMEM((_CHUNK, _DE), jnp.float32),
            pltpu.VMEM((3, 128), jnp.float32),
            pltpu.SemaphoreType.DMA,
        ],
    )
    def body(t_tab_h, w_tab_h, d_tab_h, idx3_h, out_h,
             ti_v, wi_v, di_v, gbuf, pbuf, rbuf, sem):
        wid = lax.axis_index("s") * _NC + lax.axis_index("c")
        row0 = wid * _BPW
        pltpu.sync_copy(idx3_h.at[0, pl.ds(row0, _BPW)], ti_v)
        pltpu.sync_copy(idx3_h.at[1, pl.ds(row0, _BPW)], wi_v)
        pltpu.sync_copy(idx3_h.at[2, pl.ds(row0, _BPW)], di_v)

        def is_uniform(idx_v):
            vmin = idx_v[pl.ds(0, 16)]
            vmax = vmin
            for k in range(_BPW // 16):
                v = idx_v[pl.ds(k * 16, 16)]
                vmin = jnp.minimum(vmin, v)
                vmax = jnp.maximum(vmax, v)
            first = jnp.full((16,), idx_v[pl.ds(0, 16)][0], jnp.int32)
            diff = (vmin ^ first) | (vmax ^ first)
            acc = diff[0]
            for k in range(1, 16):
                acc = acc | diff[k]
            return acc == 0

        tables = (
            (d_tab_h, di_v, 0, _DD),
            (t_tab_h, ti_v, _DD, _DT),
            (w_tab_h, wi_v, _DD + _DT, _DW),
        )
        uniforms = [is_uniform(idx_v) for _, idx_v, _, _ in tables]

        fetches = [
            pltpu.async_copy(tab_h.at[idx_v.at[pl.ds(0, 1)]],
                             rbuf.at[pl.ds(n, 1)], sem)
            for n, (tab_h, idx_v, _, _) in enumerate(tables)
        ]
        for f in fetches:
            f.wait()

        for n, ((tab_h, idx_v, col0, width), uni) in enumerate(
                zip(tables, uniforms)):
            @pl.when(uni)
            def _(n=n, col0=col0, width=width):
                row = [rbuf[n, pl.ds(k * 16, 16)]
                       for k in range(width // 16)]

                def bcast(i, c):
                    for k in range(width // 16):
                        pbuf[i, pl.ds(col0 + k * 16, 16)] = row[k]
                    return c

                lax.fori_loop(0, _CHUNK, bcast, 0)

        uall = uniforms[0] & uniforms[1] & uniforms[2]

        @pl.when(uall)
        def _():
            wbs = [pltpu.async_copy(
                       pbuf, out_h.at[pl.ds(row0 + j * _CHUNK, _CHUNK)], sem)
                   for j in range(_NCH)]
            for wb in wbs:
                wb.wait()

        @pl.when(jnp.logical_not(uall))
        def _():
            for j in range(_NCH):
                for (tab_h, idx_v, col0, width), uni in zip(tables, uniforms):
                    @pl.when(jnp.logical_not(uni))
                    def _(tab_h=tab_h, idx_v=idx_v, col0=col0,
                          width=width, j=j):
                        pltpu.async_copy(
                            tab_h.at[idx_v.at[pl.ds(j * _CHUNK, _CHUNK)]],
                            gbuf, sem).wait()

                        def pack(i, c):
                            for k in range(width // 16):
                                pbuf[i, pl.ds(col0 + k * 16, 16)] = (
                                    gbuf[i, pl.ds(k * 16, 16)])
                            return c

                        lax.fori_loop(0, _CHUNK, pack, 0)
                pltpu.sync_copy(
                    pbuf, out_h.at[pl.ds(row0 + j * _CHUNK, _CHUNK)])

    return body(t_tab, w_tab, d_tab, idx3)


def _idx_body(x_ref, out_ref):
    r = lax.broadcasted_iota(jnp.int32, (8, _NF), 0)
    c = lax.broadcasted_iota(jnp.int32, (8, _NF), 1)
    sel = (r == c).astype(jnp.float32)
    prod = lax.dot_general(sel, x_ref[...], (((1,), (1,)), ((), ())),
                           precision=lax.Precision.HIGHEST,
                           preferred_element_type=jnp.float32)
    out_ref[...] = prod.astype(jnp.int32)


_idx_call = pl.pallas_call(
    _idx_body,
    grid=(_GRID,),
    in_specs=[pl.BlockSpec((_BB, _NF), lambda i: (i, 0))],
    out_specs=pl.BlockSpec((8, _BB), lambda i: (0, i)),
    out_shape=jax.ShapeDtypeStruct((8, _B), jnp.int32),
    compiler_params=pltpu.CompilerParams(
        dimension_semantics=("arbitrary",),
    ),
)


def _bdot(a, b):
    return jnp.dot(a.astype(jnp.bfloat16), b,
                   preferred_element_type=jnp.float32)


def _mlp_body(x_ref, e_ref, w1x_ref, w1e_ref, b1_ref, w2_ref, b2_ref,
              w3_ref, b3_ref, out_ref):
    h = _bdot(x_ref[...], w1x_ref[...])
    h = h + _bdot(e_ref[...], w1e_ref[...])
    h = jnp.maximum(h + b1_ref[...], 0.0)
    h = _bdot(h, w2_ref[...].astype(jnp.bfloat16))
    h = jnp.maximum(h + b2_ref[...], 0.0)
    out_ref[...] = jnp.dot(h, w3_ref[...],
                           precision=lax.Precision.HIGHEST,
                           preferred_element_type=jnp.float32) + b3_ref[...]


_mlp_call = pl.pallas_call(
    _mlp_body,
    grid=(_GRID,),
    in_specs=[
        pl.BlockSpec((_BB, _NF), lambda i: (i, 0)),
        pl.BlockSpec((_BB, _DE), lambda i: (i, 0)),
        pl.BlockSpec((_NF, _HID), lambda i: (0, 0)),
        pl.BlockSpec((_DE, _HID), lambda i: (0, 0)),
        pl.BlockSpec((1, _HID), lambda i: (0, 0)),
        pl.BlockSpec((_HID, _HID), lambda i: (0, 0)),
        pl.BlockSpec((1, _HID), lambda i: (0, 0)),
        pl.BlockSpec((_HID, 1), lambda i: (0, 0)),
        pl.BlockSpec((1, 1), lambda i: (0, 0)),
    ],
    out_specs=pl.BlockSpec((_BB, 1), lambda i: (i, 0)),
    out_shape=jax.ShapeDtypeStruct((_B, 1), jnp.float32),
    compiler_params=pltpu.CompilerParams(
        dimension_semantics=("arbitrary",),
    ),
)


def kernel(x, timeID_em, weekID_em, driverID_em, W1, b1, W2, b2, W3, b3):
    idx3 = _idx_call(x)
    emb = _sc_gather(
        jnp.pad(timeID_em, ((0, 0), (0, 128 - _DT))),
        jnp.pad(weekID_em, ((0, 0), (0, 128 - _DW))),
        driverID_em, idx3)
    W1b = W1.astype(jnp.bfloat16)
    w1x = jnp.concatenate(
        [jnp.zeros((3, _HID), jnp.bfloat16), W1b[:_NCONT]], axis=0)
    w1e = jnp.concatenate(
        [W1b[_NCONT + _DT + _DW:],
         W1b[_NCONT:_NCONT + _DT],
         W1b[_NCONT + _DT:_NCONT + _DT + _DW]],
        axis=0)
    return _mlp_call(
        x, emb, w1x, w1e,
        b1.reshape(1, _HID), W2, b2.reshape(1, _HID),
        W3, b3.reshape(1, 1))


# speedup vs baseline: 1.8625x; 1.8625x over previous
import functools

import jax
import jax.numpy as jnp
from jax import lax
from jax.experimental import pallas as pl
from jax.experimental.pallas import tpu as pltpu
from jax.experimental.pallas import tpu_sc as plsc

_B = 16384
_NF = 128
_HID = 1024
_DT, _DW, _DD = 64, 16, 128
_NCONT = _NF - 3
_DE = _DD + _DT + _DW

_NC, _NS = 2, 16
_NW = _NC * _NS
_BPW = _B // _NW
_CHUNK = 128
_NCH = _BPW // _CHUNK

_BB = 2048
_GRID = _B // _BB


def _sc_gather(t_tab, w_tab, d_tab, idx3):
    mesh = plsc.VectorSubcoreMesh(core_axis_name="c", subcore_axis_name="s")

    @functools.partial(
        pl.kernel,
        mesh=mesh,
        out_type=jax.ShapeDtypeStruct((_B, _DE), jnp.float32),
        scratch_types=[
            pltpu.VMEM((_BPW,), jnp.int32),
            pltpu.VMEM((_BPW,), jnp.int32),
            pltpu.VMEM((_BPW,), jnp.int32),
            pltpu.VMEM((_CHUNK, 128), jnp.float32),
            pltpu.VMEM((_CHUNK, _DE), jnp.float32),
            pltpu.VMEM((3, 128), jnp.float32),
            pltpu.SemaphoreType.DMA,
        ],
    )
    def body(t_tab_h, w_tab_h, d_tab_h, idx3_h, out_h,
             ti_v, wi_v, di_v, gbuf, pbuf, rbuf, sem):
        wid = lax.axis_index("s") * _NC + lax.axis_index("c")
        row0 = wid * _BPW
        pltpu.sync_copy(idx3_h.at[0, pl.ds(row0, _BPW)], ti_v)
        pltpu.sync_copy(idx3_h.at[1, pl.ds(row0, _BPW)], wi_v)
        pltpu.sync_copy(idx3_h.at[2, pl.ds(row0, _BPW)], di_v)

        def is_uniform(idx_v):
            vmin = idx_v[pl.ds(0, 16)]
            vmax = vmin
            for k in range(_BPW // 16):
                v = idx_v[pl.ds(k * 16, 16)]
                vmin = jnp.minimum(vmin, v)
                vmax = jnp.maximum(vmax, v)
            first = jnp.full((16,), idx_v[pl.ds(0, 16)][0], jnp.int32)
            diff = (vmin ^ first) | (vmax ^ first)
            acc = diff[0]
            for k in range(1, 16):
                acc = acc | diff[k]
            return acc == 0

        tables = (
            (d_tab_h, di_v, 0, _DD),
            (t_tab_h, ti_v, _DD, _DT),
            (w_tab_h, wi_v, _DD + _DT, _DW),
        )
        uniforms = [is_uniform(idx_v) for _, idx_v, _, _ in tables]

        fetches = [
            pltpu.async_copy(tab_h.at[idx_v.at[pl.ds(0, 1)]],
                             rbuf.at[pl.ds(n, 1)], sem)
            for n, (tab_h, idx_v, _, _) in enumerate(tables)
        ]
        for f in fetches:
            f.wait()

        for n, ((tab_h, idx_v, col0, width), uni) in enumerate(
                zip(tables, uniforms)):
            @pl.when(uni)
            def _(n=n, col0=col0, width=width):
                row = [rbuf[n, pl.ds(k * 16, 16)]
                       for k in range(width // 16)]

                def bcast(i, c):
                    for k in range(width // 16):
                        pbuf[i, pl.ds(col0 + k * 16, 16)] = row[k]
                    return c

                lax.fori_loop(0, _CHUNK, bcast, 0)

        uall = uniforms[0] & uniforms[1] & uniforms[2]

        @pl.when(uall)
        def _():
            wbs = [pltpu.async_copy(
                       pbuf, out_h.at[pl.ds(row0 + j * _CHUNK, _CHUNK)], sem)
                   for j in range(_NCH)]
            for wb in wbs:
                wb.wait()

        @pl.when(jnp.logical_not(uall))
        def _():
            for j in range(_NCH):
                for (tab_h, idx_v, col0, width), uni in zip(tables, uniforms):
                    @pl.when(jnp.logical_not(uni))
                    def _(tab_h=tab_h, idx_v=idx_v, col0=col0,
                          width=width, j=j):
                        pltpu.async_copy(
                            tab_h.at[idx_v.at[pl.ds(j * _CHUNK, _CHUNK)]],
                            gbuf, sem).wait()

                        def pack(i, c):
                            for k in range(width // 16):
                                pbuf[i, pl.ds(col0 + k * 16, 16)] = (
                                    gbuf[i, pl.ds(k * 16, 16)])
                            return c

                        lax.fori_loop(0, _CHUNK, pack, 0)
                pltpu.sync_copy(
                    pbuf, out_h.at[pl.ds(row0 + j * _CHUNK, _CHUNK)])

    return body(t_tab, w_tab, d_tab, idx3)


def _idx_body(x_ref, out_ref):
    r = lax.broadcasted_iota(jnp.int32, (8, _NF), 0)
    c = lax.broadcasted_iota(jnp.int32, (8, _NF), 1)
    sel = (r == c).astype(jnp.float32)
    prod = lax.dot_general(sel, x_ref[...], (((1,), (1,)), ((), ())),
                           precision=lax.Precision.HIGHEST,
                           preferred_element_type=jnp.float32)
    out_ref[...] = prod.astype(jnp.int32)


_idx_call = pl.pallas_call(
    _idx_body,
    grid=(_GRID,),
    in_specs=[pl.BlockSpec((_BB, _NF), lambda i: (i, 0))],
    out_specs=pl.BlockSpec((8, _BB), lambda i: (0, i)),
    out_shape=jax.ShapeDtypeStruct((8, _B), jnp.int32),
    compiler_params=pltpu.CompilerParams(
        dimension_semantics=("arbitrary",),
    ),
)


def _bdot(a, b):
    return jnp.dot(a.astype(jnp.bfloat16), b,
                   preferred_element_type=jnp.float32)


def _mlp_body(x_ref, e_ref, w1x_ref, w1e_ref, b1_ref, w2_ref, b2_ref,
              w3_ref, b3_ref, out_ref):
    h = _bdot(x_ref[...], w1x_ref[...])
    h = h + _bdot(e_ref[...], w1e_ref[...])
    h = jnp.maximum(h + b1_ref[...], 0.0)
    h = _bdot(h, w2_ref[...].astype(jnp.bfloat16))
    h = jnp.maximum(h + b2_ref[...], 0.0)
    out_ref[...] = jnp.sum(h * w3_ref[...], axis=1, keepdims=True) + b3_ref[...]


_mlp_call = pl.pallas_call(
    _mlp_body,
    grid=(_GRID,),
    in_specs=[
        pl.BlockSpec((_BB, _NF), lambda i: (i, 0)),
        pl.BlockSpec((_BB, _DE), lambda i: (i, 0)),
        pl.BlockSpec((_NF, _HID), lambda i: (0, 0)),
        pl.BlockSpec((_DE, _HID), lambda i: (0, 0)),
        pl.BlockSpec((1, _HID), lambda i: (0, 0)),
        pl.BlockSpec((_HID, _HID), lambda i: (0, 0)),
        pl.BlockSpec((1, _HID), lambda i: (0, 0)),
        pl.BlockSpec((1, _HID), lambda i: (0, 0)),
        pl.BlockSpec((1, 1), lambda i: (0, 0)),
    ],
    out_specs=pl.BlockSpec((_BB, 1), lambda i: (i, 0)),
    out_shape=jax.ShapeDtypeStruct((_B, 1), jnp.float32),
    compiler_params=pltpu.CompilerParams(
        dimension_semantics=("arbitrary",),
    ),
)


def kernel(x, timeID_em, weekID_em, driverID_em, W1, b1, W2, b2, W3, b3):
    idx3 = _idx_call(x)
    emb = _sc_gather(
        jnp.pad(timeID_em, ((0, 0), (0, 128 - _DT))),
        jnp.pad(weekID_em, ((0, 0), (0, 128 - _DW))),
        driverID_em, idx3)
    W1b = W1.astype(jnp.bfloat16)
    w1x = jnp.concatenate(
        [jnp.zeros((3, _HID), jnp.bfloat16), W1b[:_NCONT]], axis=0)
    w1e = jnp.concatenate(
        [W1b[_NCONT + _DT + _DW:],
         W1b[_NCONT:_NCONT + _DT],
         W1b[_NCONT + _DT:_NCONT + _DT + _DW]],
        axis=0)
    return _mlp_call(
        x, emb, w1x, w1e,
        b1.reshape(1, _HID), W2, b2.reshape(1, _HID),
        W3.reshape(1, _HID), b3.reshape(1, 1))


# speedup vs baseline: 1.8809x; 1.0099x over previous
import functools

import jax
import jax.numpy as jnp
from jax import lax
from jax.experimental import pallas as pl
from jax.experimental.pallas import tpu as pltpu
from jax.experimental.pallas import tpu_sc as plsc

_B = 16384
_NF = 128
_HID = 1024
_DT, _DW, _DD = 64, 16, 128
_NCONT = _NF - 3
_DE = _DD + _DT + _DW

_NC, _NS = 2, 16
_NW = _NC * _NS
_BPW = _B // _NW
_CHUNK = 128
_NCH = _BPW // _CHUNK

_BB = 4096
_GRID = _B // _BB


def _sc_gather(t_tab, w_tab, d_tab, idx3):
    mesh = plsc.VectorSubcoreMesh(core_axis_name="c", subcore_axis_name="s")

    @functools.partial(
        pl.kernel,
        mesh=mesh,
        out_type=jax.ShapeDtypeStruct((_B, _DE), jnp.float32),
        scratch_types=[
            pltpu.VMEM((_BPW,), jnp.int32),
            pltpu.VMEM((_BPW,), jnp.int32),
            pltpu.VMEM((_BPW,), jnp.int32),
            pltpu.VMEM((_CHUNK, 128), jnp.float32),
            pltpu.VMEM((_CHUNK, _DE), jnp.float32),
            pltpu.VMEM((3, 128), jnp.float32),
            pltpu.SemaphoreType.DMA,
        ],
    )
    def body(t_tab_h, w_tab_h, d_tab_h, idx3_h, out_h,
             ti_v, wi_v, di_v, gbuf, pbuf, rbuf, sem):
        wid = lax.axis_index("s") * _NC + lax.axis_index("c")
        row0 = wid * _BPW
        pltpu.sync_copy(idx3_h.at[0, pl.ds(row0, _BPW)], ti_v)
        pltpu.sync_copy(idx3_h.at[1, pl.ds(row0, _BPW)], wi_v)
        pltpu.sync_copy(idx3_h.at[2, pl.ds(row0, _BPW)], di_v)

        def is_uniform(idx_v):
            vmin = idx_v[pl.ds(0, 16)]
            vmax = vmin
            for k in range(_BPW // 16):
                v = idx_v[pl.ds(k * 16, 16)]
                vmin = jnp.minimum(vmin, v)
                vmax = jnp.maximum(vmax, v)
            first = jnp.full((16,), idx_v[pl.ds(0, 16)][0], jnp.int32)
            diff = (vmin ^ first) | (vmax ^ first)
            acc = diff[0]
            for k in range(1, 16):
                acc = acc | diff[k]
            return acc == 0

        tables = (
            (d_tab_h, di_v, 0, _DD),
            (t_tab_h, ti_v, _DD, _DT),
            (w_tab_h, wi_v, _DD + _DT, _DW),
        )
        uniforms = [is_uniform(idx_v) for _, idx_v, _, _ in tables]

        fetches = [
            pltpu.async_copy(tab_h.at[idx_v.at[pl.ds(0, 1)]],
                             rbuf.at[pl.ds(n, 1)], sem)
            for n, (tab_h, idx_v, _, _) in enumerate(tables)
        ]
        for f in fetches:
            f.wait()

        for n, ((tab_h, idx_v, col0, width), uni) in enumerate(
                zip(tables, uniforms)):
            @pl.when(uni)
            def _(n=n, col0=col0, width=width):
                row = [rbuf[n, pl.ds(k * 16, 16)]
                       for k in range(width // 16)]

                def bcast(i, c):
                    for k in range(width // 16):
                        pbuf[i, pl.ds(col0 + k * 16, 16)] = row[k]
                    return c

                lax.fori_loop(0, _CHUNK, bcast, 0)

        uall = uniforms[0] & uniforms[1] & uniforms[2]

        @pl.when(uall)
        def _():
            wbs = [pltpu.async_copy(
                       pbuf, out_h.at[pl.ds(row0 + j * _CHUNK, _CHUNK)], sem)
                   for j in range(_NCH)]
            for wb in wbs:
                wb.wait()

        @pl.when(jnp.logical_not(uall))
        def _():
            for j in range(_NCH):
                for (tab_h, idx_v, col0, width), uni in zip(tables, uniforms):
                    @pl.when(jnp.logical_not(uni))
                    def _(tab_h=tab_h, idx_v=idx_v, col0=col0,
                          width=width, j=j):
                        pltpu.async_copy(
                            tab_h.at[idx_v.at[pl.ds(j * _CHUNK, _CHUNK)]],
                            gbuf, sem).wait()

                        def pack(i, c):
                            for k in range(width // 16):
                                pbuf[i, pl.ds(col0 + k * 16, 16)] = (
                                    gbuf[i, pl.ds(k * 16, 16)])
                            return c

                        lax.fori_loop(0, _CHUNK, pack, 0)
                pltpu.sync_copy(
                    pbuf, out_h.at[pl.ds(row0 + j * _CHUNK, _CHUNK)])

    return body(t_tab, w_tab, d_tab, idx3)


def _idx_body(x_ref, out_ref):
    r = lax.broadcasted_iota(jnp.int32, (8, _NF), 0)
    c = lax.broadcasted_iota(jnp.int32, (8, _NF), 1)
    sel = (r == c).astype(jnp.float32)
    prod = lax.dot_general(sel, x_ref[...], (((1,), (1,)), ((), ())),
                           precision=lax.Precision.HIGHEST,
                           preferred_element_type=jnp.float32)
    out_ref[...] = prod.astype(jnp.int32)


_idx_call = pl.pallas_call(
    _idx_body,
    grid=(_GRID,),
    in_specs=[pl.BlockSpec((_BB, _NF), lambda i: (i, 0))],
    out_specs=pl.BlockSpec((8, _BB), lambda i: (0, i)),
    out_shape=jax.ShapeDtypeStruct((8, _B), jnp.int32),
    compiler_params=pltpu.CompilerParams(
        dimension_semantics=("arbitrary",),
    ),
)


def _bdot(a, b):
    return jnp.dot(a.astype(jnp.bfloat16), b,
                   preferred_element_type=jnp.float32)


def _mlp_body(x_ref, e_ref, w1x_ref, w1e_ref, b1_ref, w2_ref, b2_ref,
              w3_ref, b3_ref, out_ref):
    h = _bdot(x_ref[...], w1x_ref[...])
    h = h + _bdot(e_ref[...], w1e_ref[...])
    h = jnp.maximum(h + b1_ref[...], 0.0)
    h = _bdot(h, w2_ref[...].astype(jnp.bfloat16))
    h = jnp.maximum(h + b2_ref[...], 0.0)
    out_ref[...] = jnp.sum(h * w3_ref[...], axis=1, keepdims=True) + b3_ref[...]


_mlp_call = pl.pallas_call(
    _mlp_body,
    grid=(_GRID,),
    in_specs=[
        pl.BlockSpec((_BB, _NF), lambda i: (i, 0)),
        pl.BlockSpec((_BB, _DE), lambda i: (i, 0)),
        pl.BlockSpec((_NF, _HID), lambda i: (0, 0)),
        pl.BlockSpec((_DE, _HID), lambda i: (0, 0)),
        pl.BlockSpec((1, _HID), lambda i: (0, 0)),
        pl.BlockSpec((_HID, _HID), lambda i: (0, 0)),
        pl.BlockSpec((1, _HID), lambda i: (0, 0)),
        pl.BlockSpec((1, _HID), lambda i: (0, 0)),
        pl.BlockSpec((1, 1), lambda i: (0, 0)),
    ],
    out_specs=pl.BlockSpec((_BB, 1), lambda i: (i, 0)),
    out_shape=jax.ShapeDtypeStruct((_B, 1), jnp.float32),
    compiler_params=pltpu.CompilerParams(
        dimension_semantics=("arbitrary",),
    ),
)


def kernel(x, timeID_em, weekID_em, driverID_em, W1, b1, W2, b2, W3, b3):
    idx3 = _idx_call(x)
    emb = _sc_gather(
        jnp.pad(timeID_em, ((0, 0), (0, 128 - _DT))),
        jnp.pad(weekID_em, ((0, 0), (0, 128 - _DW))),
        driverID_em, idx3)
    W1b = W1.astype(jnp.bfloat16)
    w1x = jnp.concatenate(
        [jnp.zeros((3, _HID), jnp.bfloat16), W1b[:_NCONT]], axis=0)
    w1e = jnp.concatenate(
        [W1b[_NCONT + _DT + _DW:],
         W1b[_NCONT:_NCONT + _DT],
         W1b[_NCONT + _DT:_NCONT + _DT + _DW]],
        axis=0)
    return _mlp_call(
        x, emb, w1x, w1e,
        b1.reshape(1, _HID), W2, b2.reshape(1, _HID),
        W3.reshape(1, _HID), b3.reshape(1, 1))
